# baseline (speedup 1.0000x reference)
import jax
import jax.numpy as jnp
from jax import lax
from jax.experimental import pallas as pl
from jax.experimental.pallas import tpu as pltpu
from jax.experimental.pallas import tpu_sc as plsc

B = 4096
D = 64
LANES = 16
NC, NS = 2, 16
NW = NC * NS
BPW = B // NW
LW = 200
LG = 50
LGG = 56
LP = 208
GP = 64
DC = D // LANES
H = 150
HP = 160
HQ = HP // LANES
O = 3
OP = 16

NBUF = 2
GROUPS = BPW // NBUF

_PARAMS = pltpu.CompilerParams(use_tc_tiling_on_sc=False,
                               needs_layout_passes=False)


def _graph_body(graph_table, gflat_hbm, out_hbm,
                gidx_v, gflat_v, gbuf, grow0, out_v, *sems):
    wid = lax.axis_index("s") * NC + lax.axis_index("c")
    base = wid * BPW

    def zpad(i, carry):
        gidx_v[i, pl.ds(GP - LANES, LANES)] = jnp.zeros((LANES,), jnp.int32)
        return carry

    lax.fori_loop(0, BPW, zpad, 0)
    pltpu.sync_copy(gflat_hbm.at[pl.ds(base * LG, BPW * LG)], gflat_v)
    pltpu.sync_copy(graph_table.at[pl.ds(0, 1)], grow0)

    def gredist(i, carry):
        for o in (0, 16, 32, 34):
            gidx_v[i, pl.ds(o, LANES)] = gflat_v[pl.ds(i * LG + o, LANES)]
        return carry

    lax.fori_loop(0, BPW, gredist, 0)

    def fire(i, b):
        pltpu.async_copy(graph_table.at[gidx_v.at[i, pl.ds(0, LGG)]],
                         gbuf.at[b], sems[b])

    def drain(b):
        pltpu.make_async_copy(graph_table.at[pl.ds(0, LGG)],
                              gbuf.at[b], sems[b]).wait()

    for b in range(NBUF):
        fire(b, b)

    def process(i, b):
        acc0 = jnp.zeros((LANES,), jnp.int32)
        for k in range(GP // LANES):
            chunk = gidx_v[i, pl.ds(k * LANES, LANES)]
            acc0 = acc0 + jnp.where(chunk == 0, 1, 0).astype(jnp.int32)
        n0g = jnp.sum(acc0)

        def gsum(r, accs):
            return tuple(accs[c] + gbuf[b, r, pl.ds(c * LANES, LANES)]
                         for c in range(DC))

        zeros = tuple(jnp.zeros((LANES,), jnp.float32) for _ in range(DC))
        gaccs = lax.fori_loop(0, LGG, gsum, zeros, unroll=4)

        ones = jnp.ones((LANES,), jnp.float32)
        n0g_f = jnp.full((LANES,), n0g - (GP - LG),
                         jnp.int32).astype(jnp.float32)
        inv_g = ones / jnp.maximum(jnp.float32(LG) - n0g_f, ones)
        for c in range(DC):
            sl = pl.ds(c * LANES, LANES)
            out_v[i, sl] = (gaccs[c]
                            - (n0g_f + (LGG - LG)) * grow0[0, sl]) * inv_g

    def group(g, carry):
        for b in range(NBUF):
            i = g * NBUF + b
            drain(b)
            process(i, b)

            @pl.when(i + NBUF < BPW)
            def _():
                fire(i + NBUF, b)
        return carry

    lax.fori_loop(0, GROUPS, group, 0)
    pltpu.sync_copy(out_v, out_hbm.at[pl.ds(base, BPW)])


def _word_body(word_table, widx_hbm, part_hbm,
               w1_hbm, b1_hbm, w2_hbm, b2_hbm, w3t_hbm, b3_hbm, out_hbm,
               widx_v, part_v, wbuf, wrow0,
               w1_v, b1_v, w2_v, b2_v, w3t_v, b3_v,
               comb_v, h1_v, out_v, *sems):
    wid = lax.axis_index("s") * NC + lax.axis_index("c")
    base = wid * BPW

    def zpad(i, carry):
        widx_v[i, pl.ds(LP - LANES, LANES)] = jnp.zeros((LANES,), jnp.int32)
        return carry

    lax.fori_loop(0, BPW, zpad, 0)
    pltpu.sync_copy(widx_hbm.at[pl.ds(base, BPW)],
                    widx_v.at[:, pl.ds(0, LW)])
    pltpu.sync_copy(part_hbm.at[pl.ds(base, BPW)], part_v)
    pltpu.sync_copy(word_table.at[pl.ds(0, 1)], wrow0)
    pltpu.sync_copy(w1_hbm, w1_v)
    pltpu.sync_copy(b1_hbm, b1_v)
    pltpu.sync_copy(w2_hbm, w2_v)
    pltpu.sync_copy(b2_hbm, b2_v)
    pltpu.sync_copy(w3t_hbm, w3t_v)
    pltpu.sync_copy(b3_hbm, b3_v)

    def fire(i, b):
        pltpu.async_copy(word_table.at[widx_v.at[i, pl.ds(0, LW)]],
                         wbuf.at[b], sems[b])

    def drain(b):
        pltpu.make_async_copy(word_table.at[pl.ds(0, LW)],
                              wbuf.at[b], sems[b]).wait()

    for b in range(NBUF):
        fire(b, b)

    def process(i, b):
        acc0 = jnp.zeros((LANES,), jnp.int32)
        for k in range(LP // LANES):
            chunk = widx_v[i, pl.ds(k * LANES, LANES)]
            acc0 = acc0 + jnp.where(chunk == 0, 1, 0).astype(jnp.int32)
        n0w = jnp.sum(acc0)

        def wsum(r, accs):
            return tuple(accs[c] + wbuf[b, r, pl.ds(c * LANES, LANES)]
                         for c in range(DC))

        zeros = tuple(jnp.zeros((LANES,), jnp.float32) for _ in range(DC))
        waccs = lax.fori_loop(0, LW, wsum, zeros, unroll=4)

        ones = jnp.ones((LANES,), jnp.float32)
        n0w_f = jnp.full((LANES,), n0w - (LP - LW),
                         jnp.int32).astype(jnp.float32)
        inv_w = ones / jnp.maximum(jnp.float32(LW) - n0w_f, ones)
        for c in range(DC):
            sl = pl.ds(c * LANES, LANES)
            comb_v[sl] = ((waccs[c] - n0w_f * wrow0[0, sl]) * inv_w
                          + part_v[i, sl])

        def dense(x_ref, n_in, w_ref):
            z = tuple(jnp.zeros((LANES,), jnp.float32) for _ in range(HQ))

            def lanes(accs, xc, base_k, nl):
                for l in range(nl):
                    xk = xc[l]
                    k = base_k + l
                    accs = tuple(
                        accs[q] + xk * w_ref[k, pl.ds(q * LANES, LANES)]
                        for q in range(HQ))
                return accs

            def cbody(t, accs):
                xc = x_ref[pl.ds(t * LANES, LANES)]
                return lanes(accs, xc, t * LANES, LANES)

            nch = n_in // LANES
            accs = lax.fori_loop(0, nch, cbody, z)
            rem = n_in - nch * LANES
            if rem:
                xc = x_ref[pl.ds(nch * LANES, LANES)]
                accs = lanes(accs, xc, nch * LANES, rem)
            return accs

        a1 = dense(comb_v, D, w1_v)
        for q in range(HQ):
            h = jnp.maximum(a1[q] + b1_v[pl.ds(q * LANES, LANES)], 0.0)
            h1_v[pl.ds(q * LANES, LANES)] = h

        a2 = dense(h1_v, H, w2_v)
        h2 = tuple(
            jnp.maximum(a2[q] + b2_v[pl.ds(q * LANES, LANES)], 0.0)
            for q in range(HQ))

        lane = lax.iota(jnp.int32, LANES)
        logits = b3_v[pl.ds(0, OP)]
        for j in range(O):
            acc = jnp.zeros((LANES,), jnp.float32)
            for q in range(HQ):
                acc = acc + h2[q] * w3t_v[j, pl.ds(q * LANES, LANES)]
            logits = logits + jnp.where(lane == j, jnp.sum(acc), 0.0)
        out_v[i, pl.ds(0, OP)] = logits

    def group(g, carry):
        for b in range(NBUF):
            i = g * NBUF + b
            drain(b)
            process(i, b)

            @pl.when(i + NBUF < BPW)
            def _():
                fire(i + NBUF, b)
        return carry

    lax.fori_loop(0, GROUPS, group, 0)
    pltpu.sync_copy(out_v, out_hbm.at[pl.ds(base, BPW)])


def _graph_pool(gflat, graph_table):
    mesh = plsc.VectorSubcoreMesh(core_axis_name="c", subcore_axis_name="s",
                                  num_cores=NC, num_subcores=NS)
    kern = pl.kernel(
        _graph_body,
        out_type=jax.ShapeDtypeStruct((B, D), jnp.float32),
        mesh=mesh,
        scratch_types=[
            pltpu.VMEM((BPW, GP), jnp.int32),
            pltpu.VMEM((BPW * LG,), jnp.int32),
            pltpu.VMEM((NBUF, LGG, D), jnp.float32),
            pltpu.VMEM((1, D), jnp.float32),
            pltpu.VMEM((BPW, D), jnp.float32),
        ] + [pltpu.SemaphoreType.DMA] * NBUF,
        compiler_params=_PARAMS,
    )
    return kern(graph_table, gflat)


def _word_mlp(widx, part, word_table, W1p, b1p, W2p, b2p, W3tp, b3p):
    mesh = plsc.VectorSubcoreMesh(core_axis_name="c", subcore_axis_name="s",
                                  num_cores=NC, num_subcores=NS)
    kern = pl.kernel(
        _word_body,
        out_type=jax.ShapeDtypeStruct((B, OP), jnp.float32),
        mesh=mesh,
        scratch_types=[
            pltpu.VMEM((BPW, LP), jnp.int32),
            pltpu.VMEM((BPW, D), jnp.float32),
            pltpu.VMEM((NBUF, LW, D), jnp.float32),
            pltpu.VMEM((1, D), jnp.float32),
            pltpu.VMEM((D, HP), jnp.float32),
            pltpu.VMEM((HP,), jnp.float32),
            pltpu.VMEM((H, HP), jnp.float32),
            pltpu.VMEM((HP,), jnp.float32),
            pltpu.VMEM((8, HP), jnp.float32),
            pltpu.VMEM((OP,), jnp.float32),
            pltpu.VMEM((D,), jnp.float32),
            pltpu.VMEM((HP,), jnp.float32),
            pltpu.VMEM((BPW, OP), jnp.float32),
        ] + [pltpu.SemaphoreType.DMA] * NBUF,
        compiler_params=_PARAMS,
    )
    return kern(word_table, widx, part,
                W1p, b1p, W2p, b2p, W3tp, b3p)


def kernel(input, graph, word_table, graph_table, alpha, beta,
           W1, b1, W2, b2, W3, b3):
    W1p = jnp.pad(W1, ((0, 0), (0, HP - H)))
    b1p = jnp.pad(b1, (0, HP - H))
    W2p = jnp.pad(W2, ((0, 0), (0, HP - H)))
    b2p = jnp.pad(b2, (0, HP - H))
    W3tp = jnp.pad(W3.T, ((0, 8 - O), (0, HP - H)))
    b3p = jnp.pad(b3, (0, OP - O))
    part = _graph_pool(graph.reshape(-1), graph_table)
    out = _word_mlp(input, part, word_table, W1p, b1p, W2p, b2p, W3tp, b3p)
    return out[:, :O]


# speedup vs baseline: 1.1784x; 1.1784x over previous
import jax
import jax.numpy as jnp
from jax import lax
from jax.experimental import pallas as pl
from jax.experimental.pallas import tpu as pltpu
from jax.experimental.pallas import tpu_sc as plsc

B = 4096
D = 64
LANES = 16
NC, NS = 2, 16
NW = NC * NS
BPW = B // NW
LW = 200
LG = 50
LGG = 56
LP = 208
GP = 64
DC = D // LANES
H = 150
HP = 160
HQ = HP // LANES
O = 3
OP = 16

NBUF = 2
GROUPS = BPW // NBUF


def _sc_body(word_table, graph_table, widx_hbm, gflat_hbm,
             w1_hbm, b1_hbm, w2_hbm, b2_hbm, w3t_hbm, b3_hbm, out_hbm,
             widx_v, gidx_v, gflat_v, wbuf, gbuf, wrow0, grow0,
             w1_v, b1_v, w2_v, b2_v, w3t_v, b3_v,
             comb_v, h1_v, out_v, *sems):
    wid = lax.axis_index("s") * NC + lax.axis_index("c")
    base = wid * BPW

    def zpad(i, carry):
        widx_v[i, pl.ds(LP - LANES, LANES)] = jnp.zeros((LANES,), jnp.int32)
        gidx_v[i, pl.ds(GP - LANES, LANES)] = jnp.zeros((LANES,), jnp.int32)
        return carry

    lax.fori_loop(0, BPW, zpad, 0)
    pltpu.sync_copy(widx_hbm.at[pl.ds(base, BPW)],
                    widx_v.at[:, pl.ds(0, LW)])
    pltpu.sync_copy(gflat_hbm.at[pl.ds(base * LG, BPW * LG)], gflat_v)
    pltpu.sync_copy(word_table.at[pl.ds(0, 1)], wrow0)
    pltpu.sync_copy(graph_table.at[pl.ds(0, 1)], grow0)
    pltpu.sync_copy(w1_hbm, w1_v)
    pltpu.sync_copy(b1_hbm, b1_v)
    pltpu.sync_copy(w2_hbm, w2_v)
    pltpu.sync_copy(b2_hbm, b2_v)
    pltpu.sync_copy(w3t_hbm, w3t_v)
    pltpu.sync_copy(b3_hbm, b3_v)

    def gredist(i, carry):
        for o in (0, 16, 32, 34):
            gidx_v[i, pl.ds(o, LANES)] = gflat_v[pl.ds(i * LG + o, LANES)]
        return carry

    lax.fori_loop(0, BPW, gredist, 0)

    def fire(i, b):
        pltpu.async_copy(word_table.at[widx_v.at[i, pl.ds(0, LW)]],
                         wbuf.at[b], sems[b])
        pltpu.async_copy(graph_table.at[gidx_v.at[i, pl.ds(0, LGG)]],
                         gbuf.at[b], sems[b])

    def drain(b):
        pltpu.make_async_copy(word_table.at[pl.ds(0, LW)],
                              wbuf.at[b], sems[b]).wait()
        pltpu.make_async_copy(graph_table.at[pl.ds(0, LGG)],
                              gbuf.at[b], sems[b]).wait()

    for b in range(NBUF):
        fire(b, b)

    def process(i, b):
        def count_zeros(idx_v, nchunks):
            acc = jnp.zeros((LANES,), jnp.int32)
            for k in range(nchunks):
                chunk = idx_v[i, pl.ds(k * LANES, LANES)]
                acc = acc + jnp.where(chunk == 0, 1, 0).astype(jnp.int32)
            return jnp.sum(acc)

        n0w = count_zeros(widx_v, LP // LANES)
        n0g = count_zeros(gidx_v, GP // LANES)

        def wsum(r, accs):
            return tuple(accs[c] + wbuf[b, r, pl.ds(c * LANES, LANES)]
                         for c in range(DC))

        def gsum(r, accs):
            return tuple(accs[c] + gbuf[b, r, pl.ds(c * LANES, LANES)]
                         for c in range(DC))

        zeros = tuple(jnp.zeros((LANES,), jnp.float32) for _ in range(DC))
        waccs = lax.fori_loop(0, LW, wsum, zeros, unroll=4)
        gaccs = lax.fori_loop(0, LGG, gsum, zeros, unroll=4)

        ones = jnp.ones((LANES,), jnp.float32)
        n0w_f = jnp.full((LANES,), n0w - (LP - LW),
                         jnp.int32).astype(jnp.float32)
        n0g_f = jnp.full((LANES,), n0g - (GP - LG),
                         jnp.int32).astype(jnp.float32)
        inv_w = ones / jnp.maximum(jnp.float32(LW) - n0w_f, ones)
        inv_g = ones / jnp.maximum(jnp.float32(LG) - n0g_f, ones)
        for c in range(DC):
            sl = pl.ds(c * LANES, LANES)
            mw = (waccs[c] - n0w_f * wrow0[0, sl]) * inv_w
            mg = (gaccs[c] - (n0g_f + (LGG - LG)) * grow0[0, sl]) * inv_g
            comb_v[sl] = mw + mg

        def dense(x_ref, n_in, w_ref):
            z = tuple(jnp.zeros((LANES,), jnp.float32) for _ in range(HQ))

            def lanes(accs, xc, base_k, nl):
                for l in range(nl):
                    xk = xc[l]
                    k = base_k + l
                    accs = tuple(
                        accs[q] + xk * w_ref[k, pl.ds(q * LANES, LANES)]
                        for q in range(HQ))
                return accs

            def cbody(t, accs):
                xc = x_ref[pl.ds(t * LANES, LANES)]
                return lanes(accs, xc, t * LANES, LANES)

            nch = n_in // LANES
            accs = lax.fori_loop(0, nch, cbody, z)
            rem = n_in - nch * LANES
            if rem:
                xc = x_ref[pl.ds(nch * LANES, LANES)]
                accs = lanes(accs, xc, nch * LANES, rem)
            return accs

        a1 = dense(comb_v, D, w1_v)
        for q in range(HQ):
            h = jnp.maximum(a1[q] + b1_v[pl.ds(q * LANES, LANES)], 0.0)
            h1_v[pl.ds(q * LANES, LANES)] = h

        a2 = dense(h1_v, H, w2_v)
        h2 = tuple(
            jnp.maximum(a2[q] + b2_v[pl.ds(q * LANES, LANES)], 0.0)
            for q in range(HQ))

        lane = lax.iota(jnp.int32, LANES)
        logits = b3_v[pl.ds(0, OP)]
        for j in range(O):
            acc = jnp.zeros((LANES,), jnp.float32)
            for q in range(HQ):
                acc = acc + h2[q] * w3t_v[j, pl.ds(q * LANES, LANES)]
            logits = logits + jnp.where(lane == j, jnp.sum(acc), 0.0)
        out_v[i, pl.ds(0, OP)] = logits

    def group(g, carry):
        for b in range(NBUF):
            i = g * NBUF + b
            drain(b)
            process(i, b)

            @pl.when(i + NBUF < BPW)
            def _():
                fire(i + NBUF, b)
        return carry

    lax.fori_loop(0, GROUPS, group, 0)
    pltpu.sync_copy(out_v, out_hbm.at[pl.ds(base, BPW)])


def _sc_classify(widx, gflat, word_table, graph_table,
                 W1p, b1p, W2p, b2p, W3tp, b3p):
    mesh = plsc.VectorSubcoreMesh(core_axis_name="c", subcore_axis_name="s",
                                  num_cores=NC, num_subcores=NS)
    kern = pl.kernel(
        _sc_body,
        out_type=jax.ShapeDtypeStruct((B, OP), jnp.float32),
        mesh=mesh,
        scratch_types=[
            pltpu.VMEM((BPW, LP), jnp.int32),
            pltpu.VMEM((BPW, GP), jnp.int32),
            pltpu.VMEM((BPW * LG,), jnp.int32),
            pltpu.VMEM((NBUF, LW, D), jnp.float32),
            pltpu.VMEM((NBUF, LGG, D), jnp.float32),
            pltpu.VMEM((1, D), jnp.float32),
            pltpu.VMEM((1, D), jnp.float32),
            pltpu.VMEM((D, HP), jnp.float32),
            pltpu.VMEM((HP,), jnp.float32),
            pltpu.VMEM((H, HP), jnp.float32),
            pltpu.VMEM((HP,), jnp.float32),
            pltpu.VMEM((8, HP), jnp.float32),
            pltpu.VMEM((OP,), jnp.float32),
            pltpu.VMEM((D,), jnp.float32),
            pltpu.VMEM((HP,), jnp.float32),
            pltpu.VMEM((BPW, OP), jnp.float32),
        ] + [pltpu.SemaphoreType.DMA] * NBUF,
        compiler_params=pltpu.CompilerParams(use_tc_tiling_on_sc=False,
                                             needs_layout_passes=False),
    )
    return kern(word_table, graph_table, widx, gflat,
                W1p, b1p, W2p, b2p, W3tp, b3p)


def kernel(input, graph, word_table, graph_table, alpha, beta,
           W1, b1, W2, b2, W3, b3):
    W1p = jnp.pad(W1, ((0, 0), (0, HP - H)))
    b1p = jnp.pad(b1, (0, HP - H))
    W2p = jnp.pad(W2, ((0, 0), (0, HP - H)))
    b2p = jnp.pad(b2, (0, HP - H))
    W3tp = jnp.pad(W3.T, ((0, 8 - O), (0, HP - H)))
    b3p = jnp.pad(b3, (0, OP - O))
    out = _sc_classify(input, graph.reshape(-1), word_table, graph_table,
                       W1p, b1p, W2p, b2p, W3tp, b3p)
    return out[:, :O]


# speedup vs baseline: 1.1808x; 1.0020x over previous
import jax
import jax.numpy as jnp
from jax import lax
from jax.experimental import pallas as pl
from jax.experimental.pallas import tpu as pltpu
from jax.experimental.pallas import tpu_sc as plsc

B = 4096
D = 64
LANES = 16
NC, NS = 2, 16
NW = NC * NS
BPW = B // NW
LW = 200
LG = 50
LGG = 56
LP = 208
GP = 64
DC = D // LANES
H = 150
HP = 160
HQ = HP // LANES
O = 3
OP = 16

NBUF = 2
GROUPS = BPW // NBUF


def _sc_body(word_table, graph_table, widx_hbm, gflat_hbm,
             w1_hbm, b1_hbm, w2_hbm, b2_hbm, w3t_hbm, b3_hbm, out_hbm,
             widx_v, gidx_v, gflat_v, wbuf, gbuf, wrow0, grow0,
             w1_v, b1_v, w2_v, b2_v, w3t_v, b3_v,
             comb_v, h1_v, out_v, *sems):
    wid = lax.axis_index("s") * NC + lax.axis_index("c")
    base = wid * BPW

    def zpad(i, carry):
        widx_v[i, pl.ds(LP - LANES, LANES)] = jnp.zeros((LANES,), jnp.int32)
        gidx_v[i, pl.ds(GP - LANES, LANES)] = jnp.zeros((LANES,), jnp.int32)
        return carry

    lax.fori_loop(0, BPW, zpad, 0)
    pltpu.sync_copy(widx_hbm.at[pl.ds(base, BPW)],
                    widx_v.at[:, pl.ds(0, LW)])
    pltpu.sync_copy(gflat_hbm.at[pl.ds(base * LG, BPW * LG)], gflat_v)
    pltpu.sync_copy(word_table.at[pl.ds(0, 1)], wrow0)
    pltpu.sync_copy(graph_table.at[pl.ds(0, 1)], grow0)

    def gredist(i, carry):
        for o in (0, 16, 32, 34):
            gidx_v[i, pl.ds(o, LANES)] = gflat_v[pl.ds(i * LG + o, LANES)]
        return carry

    lax.fori_loop(0, BPW, gredist, 0)

    LC0 = 104
    LC1 = LW - LC0

    def fire(i, b):
        pltpu.async_copy(word_table.at[widx_v.at[i, pl.ds(0, LC0)]],
                         wbuf.at[b, pl.ds(0, LC0)], sems[b])
        pltpu.async_copy(word_table.at[widx_v.at[i, pl.ds(LC0, LC1)]],
                         wbuf.at[b, pl.ds(LC0, LC1)], sems[b])
        pltpu.async_copy(graph_table.at[gidx_v.at[i, pl.ds(0, LGG)]],
                         gbuf.at[b], sems[b])

    def drain(b):
        pltpu.make_async_copy(word_table.at[pl.ds(0, LW)],
                              wbuf.at[b], sems[b]).wait()
        pltpu.make_async_copy(graph_table.at[pl.ds(0, LGG)],
                              gbuf.at[b], sems[b]).wait()

    for b in range(NBUF):
        fire(b, b)

    pltpu.sync_copy(w1_hbm, w1_v)
    pltpu.sync_copy(b1_hbm, b1_v)
    pltpu.sync_copy(w2_hbm, w2_v)
    pltpu.sync_copy(b2_hbm, b2_v)
    pltpu.sync_copy(w3t_hbm, w3t_v)
    pltpu.sync_copy(b3_hbm, b3_v)

    def process(i, b):
        def count_zeros(idx_v, nchunks):
            acc = jnp.zeros((LANES,), jnp.int32)
            for k in range(nchunks):
                chunk = idx_v[i, pl.ds(k * LANES, LANES)]
                acc = acc + jnp.where(chunk == 0, 1, 0).astype(jnp.int32)
            return jnp.sum(acc)

        n0w = count_zeros(widx_v, LP // LANES)
        n0g = count_zeros(gidx_v, GP // LANES)

        def wsum(r, accs):
            return tuple(accs[c] + wbuf[b, r, pl.ds(c * LANES, LANES)]
                         for c in range(DC))

        def gsum(r, accs):
            return tuple(accs[c] + gbuf[b, r, pl.ds(c * LANES, LANES)]
                         for c in range(DC))

        zeros = tuple(jnp.zeros((LANES,), jnp.float32) for _ in range(DC))
        waccs = lax.fori_loop(0, LW, wsum, zeros, unroll=4)
        gaccs = lax.fori_loop(0, LGG, gsum, zeros, unroll=4)

        ones = jnp.ones((LANES,), jnp.float32)
        n0w_f = jnp.full((LANES,), n0w - (LP - LW),
                         jnp.int32).astype(jnp.float32)
        n0g_f = jnp.full((LANES,), n0g - (GP - LG),
                         jnp.int32).astype(jnp.float32)
        inv_w = ones / jnp.maximum(jnp.float32(LW) - n0w_f, ones)
        inv_g = ones / jnp.maximum(jnp.float32(LG) - n0g_f, ones)
        for c in range(DC):
            sl = pl.ds(c * LANES, LANES)
            mw = (waccs[c] - n0w_f * wrow0[0, sl]) * inv_w
            mg = (gaccs[c] - (n0g_f + (LGG - LG)) * grow0[0, sl]) * inv_g
            comb_v[sl] = mw + mg

        def dense(x_ref, n_in, w_ref):
            z = tuple(jnp.zeros((LANES,), jnp.float32) for _ in range(HQ))

            def lanes(accs, xc, base_k, nl):
                for l in range(nl):
                    xk = xc[l]
                    k = base_k + l
                    accs = tuple(
                        accs[q] + xk * w_ref[k, pl.ds(q * LANES, LANES)]
                        for q in range(HQ))
                return accs

            def cbody(t, accs):
                xc = x_ref[pl.ds(t * LANES, LANES)]
                return lanes(accs, xc, t * LANES, LANES)

            nch = n_in // LANES
            accs = lax.fori_loop(0, nch, cbody, z)
            rem = n_in - nch * LANES
            if rem:
                xc = x_ref[pl.ds(nch * LANES, LANES)]
                accs = lanes(accs, xc, nch * LANES, rem)
            return accs

        a1 = dense(comb_v, D, w1_v)
        for q in range(HQ):
            h = jnp.maximum(a1[q] + b1_v[pl.ds(q * LANES, LANES)], 0.0)
            h1_v[pl.ds(q * LANES, LANES)] = h

        a2 = dense(h1_v, H, w2_v)
        h2 = tuple(
            jnp.maximum(a2[q] + b2_v[pl.ds(q * LANES, LANES)], 0.0)
            for q in range(HQ))

        lane = lax.iota(jnp.int32, LANES)
        logits = b3_v[pl.ds(0, OP)]
        for j in range(O):
            acc = jnp.zeros((LANES,), jnp.float32)
            for q in range(HQ):
                acc = acc + h2[q] * w3t_v[j, pl.ds(q * LANES, LANES)]
            logits = logits + jnp.where(lane == j, jnp.sum(acc), 0.0)
        out_v[i, pl.ds(0, OP)] = logits

    def group(g, carry):
        for b in range(NBUF):
            i = g * NBUF + b
            drain(b)
            process(i, b)

            @pl.when(i + NBUF < BPW)
            def _():
                fire(i + NBUF, b)
        return carry

    lax.fori_loop(0, GROUPS, group, 0)
    pltpu.sync_copy(out_v, out_hbm.at[pl.ds(base, BPW)])


def _sc_classify(widx, gflat, word_table, graph_table,
                 W1p, b1p, W2p, b2p, W3tp, b3p):
    mesh = plsc.VectorSubcoreMesh(core_axis_name="c", subcore_axis_name="s",
                                  num_cores=NC, num_subcores=NS)
    kern = pl.kernel(
        _sc_body,
        out_type=jax.ShapeDtypeStruct((B, OP), jnp.float32),
        mesh=mesh,
        scratch_types=[
            pltpu.VMEM((BPW, LP), jnp.int32),
            pltpu.VMEM((BPW, GP), jnp.int32),
            pltpu.VMEM((BPW * LG,), jnp.int32),
            pltpu.VMEM((NBUF, LW, D), jnp.float32),
            pltpu.VMEM((NBUF, LGG, D), jnp.float32),
            pltpu.VMEM((1, D), jnp.float32),
            pltpu.VMEM((1, D), jnp.float32),
            pltpu.VMEM((D, HP), jnp.float32),
            pltpu.VMEM((HP,), jnp.float32),
            pltpu.VMEM((H, HP), jnp.float32),
            pltpu.VMEM((HP,), jnp.float32),
            pltpu.VMEM((8, HP), jnp.float32),
            pltpu.VMEM((OP,), jnp.float32),
            pltpu.VMEM((D,), jnp.float32),
            pltpu.VMEM((HP,), jnp.float32),
            pltpu.VMEM((BPW, OP), jnp.float32),
        ] + [pltpu.SemaphoreType.DMA] * NBUF,
        compiler_params=pltpu.CompilerParams(use_tc_tiling_on_sc=False,
                                             needs_layout_passes=False),
    )
    return kern(word_table, graph_table, widx, gflat,
                W1p, b1p, W2p, b2p, W3tp, b3p)


def kernel(input, graph, word_table, graph_table, alpha, beta,
           W1, b1, W2, b2, W3, b3):
    W1p = jnp.pad(W1, ((0, 0), (0, HP - H)))
    b1p = jnp.pad(b1, (0, HP - H))
    W2p = jnp.pad(W2, ((0, 0), (0, HP - H)))
    b2p = jnp.pad(b2, (0, HP - H))
    W3tp = jnp.pad(W3.T, ((0, 8 - O), (0, HP - H)))
    b3p = jnp.pad(b3, (0, OP - O))
    out = _sc_classify(input, graph.reshape(-1), word_table, graph_table,
                       W1p, b1p, W2p, b2p, W3tp, b3p)
    return out[:, :O]
